# baseline (speedup 1.0000x reference)
import functools

import jax
import jax.numpy as jnp
from jax import lax
from jax.experimental import pallas as pl
from jax.experimental.pallas import tpu as pltpu
from jax.experimental.pallas import tpu_sc as plsc

VOCAB = 1000001
DIM = 64
L = 200
B = 4096

NUM_CORES = 2
NUM_SUBCORES = 16
NW = NUM_CORES * NUM_SUBCORES
CPW = B // NW

VB = 5000
FB = VB * 128
PB = 100
HALF = (VOCAB - 1) // 2


def _tvec_body(tab_ref, w2_ref, b_ref, te_ref, to_ref):
    tb = tab_ref[...].reshape(VB, 128)
    w2 = w2_ref[...]
    s = jax.lax.dot_general(w2, tb, (((1,), (1,)), ((), ())),
                            preferred_element_type=jnp.float32)
    s = s + b_ref[0]
    te_ref[...] = s[0:1].reshape(1, 1, VB)
    to_ref[...] = s[1:2].reshape(1, 1, VB)


def _tvec(tflat, w2, bl):
    te, to = pl.pallas_call(
        _tvec_body,
        grid=(PB,),
        in_specs=[
            pl.BlockSpec((FB,), lambda i: (i,)),
            pl.BlockSpec((2, 128), lambda i: (0, 0)),
            pl.BlockSpec(memory_space=pltpu.SMEM),
        ],
        out_specs=[pl.BlockSpec((1, 1, VB), lambda i: (i, 0, 0))] * 2,
        out_shape=[jax.ShapeDtypeStruct((PB, 1, VB), jnp.float32)] * 2,
    )(tflat, w2, bl)
    return jnp.concatenate([te.reshape(-1), to.reshape(-1)])


_CHUNK = 8
_NCHUNK = L // _CHUNK


def _sc_pool_body(t_hbm, x_hbm, out_hbm, idx_v, s_v, o_v, sem):
    wid = lax.axis_index("s") * NUM_CORES + lax.axis_index("c")
    base = wid * CPW
    pltpu.sync_copy(x_hbm.at[:, pl.ds(base, CPW)], idx_v)

    def remap_row(l, _):
        for g in range(CPW // 16):
            v = idx_v[l, pl.ds(g * 16, 16)]
            gi = (v >> 1) + (v & 1) * HALF
            gi = jnp.where(v == VOCAB - 1, VOCAB - 1, gi)
            idx_v[l, pl.ds(g * 16, 16)] = gi
        return _

    lax.fori_loop(0, L, remap_row, 0, unroll=False)

    def fire_drain(c, _):
        for i in range(_CHUNK):
            l = c * _CHUNK + i
            pltpu.async_copy(t_hbm.at[idx_v.at[l]], s_v.at[l], sem)
        for i in range(_CHUNK):
            l = c * _CHUNK + i
            pltpu.make_async_copy(t_hbm.at[idx_v.at[l]], s_v.at[l], sem).wait()
        return _

    lax.fori_loop(0, _NCHUNK, fire_drain, 0, unroll=False)

    for jg in range(CPW // 16):
        def add_row(l, acc):
            return acc + s_v[l, pl.ds(jg * 16, 16)]
        acc = lax.fori_loop(0, L, add_row, jnp.zeros((16,), jnp.float32))
        o_v[pl.ds(jg * 16, 16)] = acc

    pltpu.sync_copy(o_v, out_hbm.at[pl.ds(base, CPW)])


@functools.lru_cache(maxsize=1)
def _sc_pool():
    return pl.kernel(
        _sc_pool_body,
        out_type=jax.ShapeDtypeStruct((B,), jnp.float32),
        mesh=plsc.VectorSubcoreMesh(core_axis_name="c", subcore_axis_name="s"),
        scratch_types=[
            pltpu.VMEM((L, CPW), jnp.int32),
            pltpu.VMEM((L, CPW), jnp.float32),
            pltpu.VMEM((CPW,), jnp.float32),
            pltpu.SemaphoreType.DMA,
        ],
    )


def kernel(x, table, W, b):
    xi = x.astype(jnp.int32)
    tflat = table.reshape(-1)
    wl = (W[0] / L).astype(jnp.float32)
    z = jnp.zeros((DIM,), jnp.float32)
    w2 = jnp.stack([jnp.concatenate([wl, z]),
                    jnp.concatenate([z, wl])])
    bl = (b / L).astype(jnp.float32)
    t_head = _tvec(tflat, w2, bl)
    t_last = (jnp.dot(table[VOCAB - 1], W[0]) + b[0]).reshape(1) / L
    t = jnp.concatenate([t_head, t_last])
    return _sc_pool()(t, xi)


# speedup vs baseline: 1.4065x; 1.4065x over previous
import functools

import jax
import jax.numpy as jnp
from jax import lax
from jax.experimental import pallas as pl
from jax.experimental.pallas import tpu as pltpu
from jax.experimental.pallas import tpu_sc as plsc

VOCAB = 1000001
DIM = 64
L = 200
B = 4096

NUM_CORES = 2
NUM_SUBCORES = 16
NW = NUM_CORES * NUM_SUBCORES
CPW = B // NW

VB = 8192
TC_GRID = -(-VOCAB // VB)


def _tvec_body(tab_ref, w_ref, b_ref, t_ref):
    w = w_ref[...]
    tb = tab_ref[...]
    s = jax.lax.dot_general(w, tb, (((1,), (1,)), ((), ())),
                            preferred_element_type=jnp.float32)
    t_ref[...] = ((s + b_ref[0]) * (1.0 / L)).reshape(1, 1, VB)


def _tvec(table, W, b):
    return pl.pallas_call(
        _tvec_body,
        grid=(TC_GRID,),
        in_specs=[
            pl.BlockSpec((VB, DIM), lambda i: (i, 0)),
            pl.BlockSpec((1, DIM), lambda i: (0, 0)),
            pl.BlockSpec(memory_space=pltpu.SMEM),
        ],
        out_specs=pl.BlockSpec((1, 1, VB), lambda i: (i, 0, 0)),
        out_shape=jax.ShapeDtypeStruct((TC_GRID, 1, VB), jnp.float32),
    )(table, W, b)


_CHUNK = 8
_NCHUNK = L // _CHUNK


def _sc_pool_body(t_hbm, x_hbm, out_hbm, idx_v, s_v, o_v, sem):
    wid = lax.axis_index("s") * NUM_CORES + lax.axis_index("c")
    base = wid * CPW
    pltpu.sync_copy(x_hbm.at[:, pl.ds(base, CPW)], idx_v)

    def fire_drain(c, _):
        for i in range(_CHUNK):
            l = c * _CHUNK + i
            pltpu.async_copy(t_hbm.at[idx_v.at[l]], s_v.at[l], sem)
        for i in range(_CHUNK):
            l = c * _CHUNK + i
            pltpu.make_async_copy(t_hbm.at[idx_v.at[l]], s_v.at[l], sem).wait()
        return _

    lax.fori_loop(0, _NCHUNK, fire_drain, 0, unroll=False)

    for jg in range(CPW // 16):
        def add_row(l, acc):
            return acc + s_v[l, pl.ds(jg * 16, 16)]
        acc = lax.fori_loop(0, L, add_row, jnp.zeros((16,), jnp.float32))
        o_v[pl.ds(jg * 16, 16)] = acc

    pltpu.sync_copy(o_v, out_hbm.at[pl.ds(base, CPW)])


@functools.lru_cache(maxsize=1)
def _sc_pool():
    return pl.kernel(
        _sc_pool_body,
        out_type=jax.ShapeDtypeStruct((B,), jnp.float32),
        mesh=plsc.VectorSubcoreMesh(core_axis_name="c", subcore_axis_name="s"),
        scratch_types=[
            pltpu.VMEM((L, CPW), jnp.int32),
            pltpu.VMEM((L, CPW), jnp.float32),
            pltpu.VMEM((CPW,), jnp.float32),
            pltpu.SemaphoreType.DMA,
        ],
    )


def kernel(x, table, W, b):
    xi = x.astype(jnp.int32)
    t = _tvec(table, W, b).reshape(TC_GRID * VB)
    return _sc_pool()(t, xi)
